# baseline (speedup 1.0000x reference)
import functools

import jax
import jax.numpy as jnp
from jax.experimental import pallas as pl
from jax.experimental.pallas import tpu as pltpu


def _gcn_block(a0_ref, a1_ref, h_ref, w_ref, b_ref, o0_ref, o1_ref):
    h = h_ref[...]
    wt = w_ref[...].T
    bb = b_ref[...]
    for a_ref, o_ref in ((a0_ref, o0_ref), (a1_ref, o1_ref)):
        a = a_ref[...]
        rs = jnp.maximum(jnp.sum(a, axis=1, keepdims=True), 1e-8)
        msg = jnp.dot(a, h, preferred_element_type=jnp.float32)
        o_ref[...] = jnp.dot(msg / rs, wt, preferred_element_type=jnp.float32) + bb


@functools.partial(jax.jit, static_argnames=("bm",))
def _gcn(H_source, A, W, b2, bm):
    n_tgt, n_src = A.shape
    d_out = W.shape[0]
    half = n_tgt // 2
    nblk = half // bm
    o0, o1 = pl.pallas_call(
        _gcn_block,
        grid=(nblk,),
        in_specs=[
            pl.BlockSpec((bm, n_src), lambda i: (i, 0)),
            pl.BlockSpec((bm, n_src), lambda i: (i + nblk, 0)),
            pl.BlockSpec((n_src, H_source.shape[1]), lambda i: (0, 0)),
            pl.BlockSpec(W.shape, lambda i: (0, 0)),
            pl.BlockSpec(b2.shape, lambda i: (0, 0)),
        ],
        out_specs=[
            pl.BlockSpec((bm, d_out), lambda i: (i, 0)),
            pl.BlockSpec((bm, d_out), lambda i: (i, 0)),
        ],
        out_shape=[
            jax.ShapeDtypeStruct((half, d_out), jnp.float32),
            jax.ShapeDtypeStruct((half, d_out), jnp.float32),
        ],
        compiler_params=pltpu.CompilerParams(
            dimension_semantics=("parallel",),
        ),
    )(A, A, H_source, W, b2)
    return jnp.concatenate([o0, o1], axis=0)


def kernel(H_source, A, W, b):
    return _gcn(H_source, A, W, b.reshape(1, -1), bm=128)
